# Initial kernel scaffold; baseline (speedup 1.0000x reference)
import functools

import jax
import jax.numpy as jnp
from jax import lax
from jax.experimental import pallas as pl
from jax.experimental.pallas import tpu as pltpu
from jax.experimental.pallas import tpu_sc as plsc

NC = 2
NS = 16
NW = NC * NS
GRP = 128
K = 8


def _make_embed(n_vocab: int, d: int, n_groups_total: int):
  g_per_w = n_groups_total // NW
  mesh = plsc.VectorSubcoreMesh(core_axis_name="c", subcore_axis_name="s")

  @functools.partial(
      pl.kernel,
      mesh=mesh,
      out_type=jax.ShapeDtypeStruct((n_groups_total * GRP, d), jnp.float32),
      scratch_types=[
          pltpu.VMEM((g_per_w, GRP), jnp.int32),
          pltpu.VMEM((K, GRP, d), jnp.float32),
          pltpu.SemaphoreType.DMA,
          pltpu.SemaphoreType.DMA,
      ],
  )
  def embed(table_hbm, idx_hbm, out_hbm, idx_v, rows_v, sem_g, sem_o):
    wid = lax.axis_index("s") * NC + lax.axis_index("c")
    gbase = wid * g_per_w
    pltpu.sync_copy(idx_hbm.at[pl.ds(gbase, g_per_w)], idx_v)

    @pl.loop(0, g_per_w, step=K)
    def _step(g0):
      for b in range(K):
        pltpu.make_async_copy(
            table_hbm.at[idx_v.at[g0 + b]], rows_v.at[b], sem_g).start()
      for b in range(K):
        pltpu.make_async_copy(
            table_hbm.at[idx_v.at[g0 + b]], rows_v.at[b], sem_g).wait()
        pltpu.make_async_copy(
            rows_v.at[b],
            out_hbm.at[pl.ds((gbase + g0 + b) * GRP, GRP)],
            sem_o).start()
      for b in range(K):
        pltpu.make_async_copy(
            rows_v.at[b],
            out_hbm.at[pl.ds((gbase + g0 + b) * GRP, GRP)],
            sem_o).wait()

  return embed


def kernel(token_ids, W):
  bt, s = token_ids.shape
  n_vocab, d = W.shape
  total = bt * s
  assert total % (NW * GRP * K) == 0
  n_groups_total = total // GRP
  idx = token_ids.reshape(n_groups_total, GRP).astype(jnp.int32)
  out = _make_embed(n_vocab, d, n_groups_total)(W, idx)
  return out.reshape(bt, s, d)


# speedup vs baseline: 1.1088x; 1.1088x over previous
import functools

import jax
import jax.numpy as jnp
from jax import lax
from jax.experimental import pallas as pl
from jax.experimental.pallas import tpu as pltpu
from jax.experimental.pallas import tpu_sc as plsc

NC = 2
NS = 16
NW = NC * NS
GRP = 128
K = 8


def _make_embed(n_vocab: int, d: int, n_groups_total: int):
  g_per_w = n_groups_total // NW
  mesh = plsc.VectorSubcoreMesh(core_axis_name="c", subcore_axis_name="s")

  @functools.partial(
      pl.kernel,
      mesh=mesh,
      out_type=jax.ShapeDtypeStruct((n_groups_total * GRP, d), jnp.float32),
      scratch_types=[
          pltpu.VMEM((g_per_w, GRP), jnp.int32),
          pltpu.VMEM((K, GRP, d), jnp.float32),
          pltpu.SemaphoreType.DMA,
          pltpu.SemaphoreType.DMA,
      ],
      compiler_params=pltpu.CompilerParams(use_tc_tiling_on_sc=False),
  )
  def embed(table_hbm, idx_hbm, out_hbm, idx_v, rows_v, sem_g, sem_o):
    wid = lax.axis_index("s") * NC + lax.axis_index("c")
    gbase = wid * g_per_w
    pltpu.sync_copy(idx_hbm.at[pl.ds(gbase, g_per_w)], idx_v)

    @pl.loop(0, g_per_w, step=K)
    def _step(g0):
      for b in range(K):
        pltpu.make_async_copy(
            table_hbm.at[idx_v.at[g0 + b]], rows_v.at[b], sem_g).start()
      for b in range(K):
        pltpu.make_async_copy(
            table_hbm.at[idx_v.at[g0 + b]], rows_v.at[b], sem_g).wait()
        pltpu.make_async_copy(
            rows_v.at[b],
            out_hbm.at[pl.ds((gbase + g0 + b) * GRP, GRP)],
            sem_o).start()
      for b in range(K):
        pltpu.make_async_copy(
            rows_v.at[b],
            out_hbm.at[pl.ds((gbase + g0 + b) * GRP, GRP)],
            sem_o).wait()

  return embed


def kernel(token_ids, W):
  bt, s = token_ids.shape
  n_vocab, d = W.shape
  total = bt * s
  assert total % (NW * GRP * K) == 0
  n_groups_total = total // GRP
  idx = token_ids.reshape(n_groups_total, GRP).astype(jnp.int32)
  out = _make_embed(n_vocab, d, n_groups_total)(W, idx)
  return out.reshape(bt, s, d)


# speedup vs baseline: 1.7729x; 1.5990x over previous
import functools

import jax
import jax.numpy as jnp
from jax import lax
from jax.experimental import pallas as pl
from jax.experimental.pallas import tpu as pltpu
from jax.experimental.pallas import tpu_sc as plsc

NC = 2
NS = 16
NW = NC * NS
CHUNK = 16


def _make_embed(n_batch: int, n_seq: int, d: int):
  rows_per_w = n_batch // NW
  n_chunks = rows_per_w // CHUNK
  mesh = plsc.VectorSubcoreMesh(core_axis_name="c", subcore_axis_name="s")

  @functools.partial(
      pl.kernel,
      mesh=mesh,
      out_type=jax.ShapeDtypeStruct((n_batch, n_seq, d), jnp.float32),
      scratch_types=[
          pltpu.VMEM((rows_per_w, n_seq), jnp.int32),
          pltpu.VMEM((2, CHUNK, n_seq, d), jnp.float32),
          pltpu.SemaphoreType.DMA,
          pltpu.SemaphoreType.DMA,
          pltpu.SemaphoreType.DMA,
      ],
      compiler_params=pltpu.CompilerParams(use_tc_tiling_on_sc=False),
  )
  def embed(table_hbm, tids_hbm, out_hbm, idx_v, chunk_v, sem_g, sem_o0,
            sem_o1):
    wid = lax.axis_index("s") * NC + lax.axis_index("c")
    base = wid * rows_per_w
    pltpu.sync_copy(tids_hbm.at[pl.ds(base, rows_per_w)], idx_v)

    def do_chunk(j, buf, sem_o):
      for i in range(CHUNK):
        pltpu.make_async_copy(
            table_hbm.at[idx_v.at[j * CHUNK + i]], chunk_v.at[buf, i],
            sem_g).start()
      for i in range(CHUNK):
        pltpu.make_async_copy(
            table_hbm.at[idx_v.at[j * CHUNK + i]], chunk_v.at[buf, i],
            sem_g).wait()
      pltpu.make_async_copy(
          chunk_v.at[buf], out_hbm.at[pl.ds(base + j * CHUNK, CHUNK)],
          sem_o).start()

    def wait_chunk(j, buf, sem_o):
      pltpu.make_async_copy(
          chunk_v.at[buf], out_hbm.at[pl.ds(base + j * CHUNK, CHUNK)],
          sem_o).wait()

    @pl.loop(0, n_chunks, step=2)
    def _step(j0):
      do_chunk(j0, 0, sem_o0)
      do_chunk(j0 + 1, 1, sem_o1)
      wait_chunk(j0, 0, sem_o0)
      wait_chunk(j0 + 1, 1, sem_o1)

  return embed


def kernel(token_ids, W):
  bt, s = token_ids.shape
  n_vocab, d = W.shape
  assert bt % (NW * CHUNK * 2) == 0
  tids = token_ids.astype(jnp.int32)
  return _make_embed(bt, s, d)(W, tids)
